# Initial kernel scaffold; baseline (speedup 1.0000x reference)
import jax
import jax.numpy as jnp
from jax.experimental import pallas as pl


def kernel(node_type, node_tactic_id, state_lm_id, edge_index, batch, type_emb, tactic_emb, state_lm_bank, W_proj, b_proj, W1_self, W1_neigh, W1_sem, b1, W2_self, W2_neigh, W2_sem, b2, Wc1, bc1, Wc2, bc2):
    raise NotImplementedError("write your pallas kernel here")


# speedup vs baseline: 1.1532x; 1.1532x over previous
import jax
import jax.numpy as jnp
from jax.experimental import pallas as pl

_N = 50000
_G = 256
_NUM_TACTICS = 1000


def _head_body(gr_ref, wc1_ref, bc1_ref, wc2_ref, bc2_ref, out_ref):
    h = jnp.maximum(gr_ref[...] @ wc1_ref[...] + bc1_ref[...], 0.0)
    out_ref[...] = h @ wc2_ref[...] + bc2_ref[...]


def _sage(x, src, dst, deg_inv, sem, W_self, W_neigh, W_sem, b):
    msg = jnp.take(x @ W_neigh, src, axis=0)
    agg = jax.ops.segment_sum(msg, dst, num_segments=_N)
    return x @ W_self + agg * deg_inv[:, None] + sem @ W_sem + b


def kernel(node_type, node_tactic_id, state_lm_id, edge_index, batch, type_emb,
           tactic_emb, state_lm_bank, W_proj, b_proj, W1_self, W1_neigh, W1_sem,
           b1, W2_self, W2_neigh, W2_sem, b2, Wc1, bc1, Wc2, bc2):
    t_type = jnp.take(type_emb, node_type, axis=0)
    shifted = jnp.clip(node_tactic_id + 1, 0, _NUM_TACTICS)
    t_tac = jnp.take(tactic_emb, shifted, axis=0)
    mask = state_lm_id >= 0
    safe_id = jnp.where(mask, state_lm_id, 0)
    proj_bank = state_lm_bank @ W_proj
    sem = jnp.where(mask[:, None], jnp.take(proj_bank, safe_id, axis=0) + b_proj, 0.0)
    x = jnp.concatenate([t_type, t_tac, sem], axis=-1)

    src, dst = edge_index[0], edge_index[1]
    deg = jax.ops.segment_sum(jnp.ones((src.shape[0],), jnp.float32), dst,
                              num_segments=_N)
    deg_inv = 1.0 / jnp.clip(deg, 1.0, None)
    x = jax.nn.relu(_sage(x, src, dst, deg_inv, sem, W1_self, W1_neigh, W1_sem, b1))
    x = jax.nn.relu(_sage(x, src, dst, deg_inv, sem, W2_self, W2_neigh, W2_sem, b2))

    sums = jax.ops.segment_sum(x, batch, num_segments=_G)
    cnt = jax.ops.segment_sum(jnp.ones((_N,), jnp.float32), batch, num_segments=_G)
    graph_repr = sums / jnp.clip(cnt, 1.0, None)[:, None]

    return pl.pallas_call(
        _head_body,
        out_shape=jax.ShapeDtypeStruct((_G, _NUM_TACTICS), jnp.float32),
    )(graph_repr, Wc1, bc1, Wc2, bc2)


# speedup vs baseline: 2.2441x; 1.9460x over previous
import functools

import jax
import jax.numpy as jnp
from jax import lax
from jax.experimental import pallas as pl
from jax.experimental.pallas import tpu as pltpu
from jax.experimental.pallas import tpu_sc as plsc

_N = 50000
_E = 800000
_G = 256
_NUM_TACTICS = 1000

_NC, _NS = 2, 16
_NP = 50176
_EP = 819200
_ER = _EP // 128
_RPT = _ER // _NS
_BB = 4
_NB = _RPT // _BB
_ZR = _NP // _NS


def _edge_body(ytall, src4, dstp, zrows, s4, sidx, didx, rows, acc, gsem):
    c = lax.axis_index("c")
    s = lax.axis_index("s")
    for jj in range(2):
        chunk = c * 2 + jj
        pltpu.sync_copy(zrows, acc.at[pl.ds(s * _ZR, _ZR)])
        plsc.subcore_barrier()

        def batch_body(b, carry):
            row0 = s * _RPT + b * _BB
            pltpu.sync_copy(src4.at[chunk, pl.ds(row0, _BB), :], sidx)
            pltpu.sync_copy(dstp.at[pl.ds(row0, _BB), :], didx)
            cps = [
                pltpu.async_copy(ytall.at[sidx.at[j]],
                                 rows.at[pl.ds(j * 128, 128)], gsem)
                for j in range(_BB)
            ]
            for cp in cps:
                cp.wait()
            for j in range(_BB):
                pltpu.sync_copy(rows.at[pl.ds(j * 128, 128)],
                                acc.at[didx.at[j]], add=True)
            return carry

        lax.fori_loop(0, _NB, batch_body, 0)
        plsc.subcore_barrier()
        pltpu.sync_copy(acc.at[pl.ds(s * _ZR, _ZR)],
                        s4.at[chunk, pl.ds(s * _ZR, _ZR), :])
        plsc.subcore_barrier()


@functools.partial(jax.jit, static_argnums=())
def _edge_segsum(ytall, src4, dstp, zrows):
    f = pl.kernel(
        _edge_body,
        out_type=jax.ShapeDtypeStruct((4, _NP, 32), jnp.float32),
        mesh=plsc.VectorSubcoreMesh(core_axis_name="c", subcore_axis_name="s",
                                    num_cores=_NC, num_subcores=_NS),
        scratch_types=[
            pltpu.VMEM((_BB, 128), jnp.int32),
            pltpu.VMEM((_BB, 128), jnp.int32),
            pltpu.VMEM((_BB * 128, 32), jnp.float32),
            pltpu.VMEM_SHARED((_NP, 32), jnp.float32),
            pltpu.SemaphoreType.DMA,
        ],
        compiler_params=pltpu.CompilerParams(use_tc_tiling_on_sc=False),
    )
    return f(ytall, src4, dstp, zrows)


_DR = _EP // 128 // (_NC * _NS)


_DB = 4
_DEPW = _EP // (_NC * _NS)
_DRW = _DEPW // 128
_DNB = _DRW // _DB
_DZR = _NP // _NS


_DH = _NP // 32
_DHT = _DH // _NS


def _deg_body(dhi, dlo, eye32, zrows_d, deg_out, hidx, lidx, rows, acc, gsem):
    c = lax.axis_index("c")
    s = lax.axis_index("s")
    w = s * _NC + c
    pltpu.sync_copy(zrows_d, acc.at[pl.ds(s * _DHT, _DHT)])
    plsc.subcore_barrier()

    def batch_body(b, carry):
        row0 = w * _DRW + b * _DB
        pltpu.sync_copy(dhi.at[pl.ds(row0, _DB), :], hidx)
        pltpu.sync_copy(dlo.at[pl.ds(row0, _DB), :], lidx)
        cps = [
            pltpu.async_copy(eye32.at[lidx.at[j]],
                             rows.at[pl.ds(j * 128, 128)], gsem)
            for j in range(_DB)
        ]
        for cp in cps:
            cp.wait()
        for j in range(_DB):
            pltpu.sync_copy(rows.at[pl.ds(j * 128, 128)],
                            acc.at[hidx.at[j]], add=True)
        return carry

    lax.fori_loop(0, _DNB, batch_body, 0)
    plsc.subcore_barrier()
    pltpu.sync_copy(acc.at[pl.ds(s * _DHT, _DHT)],
                    deg_out.at[c, pl.ds(s * _DHT, _DHT), :])


def _deg_count(dhi, dlo, eye32, zrows_d):
    f = pl.kernel(
        _deg_body,
        out_type=jax.ShapeDtypeStruct((_NC, _DH, 32), jnp.float32),
        mesh=plsc.VectorSubcoreMesh(core_axis_name="c", subcore_axis_name="s",
                                    num_cores=_NC, num_subcores=_NS),
        scratch_types=[
            pltpu.VMEM((_DB, 128), jnp.int32),
            pltpu.VMEM((_DB, 128), jnp.int32),
            pltpu.VMEM((_DB * 128, 32), jnp.float32),
            pltpu.VMEM_SHARED((_DH, 32), jnp.float32),
            pltpu.SemaphoreType.DMA,
        ],
        compiler_params=pltpu.CompilerParams(use_tc_tiling_on_sc=False),
    )
    return f(dhi, dlo, eye32, zrows_d)


def _head_body(gr_ref, wc1_ref, bc1_ref, wc2_ref, bc2_ref, out_ref):
    h = jnp.maximum(gr_ref[...] @ wc1_ref[...] + bc1_ref[...], 0.0)
    out_ref[...] = h @ wc2_ref[...] + bc2_ref[...]


def _to_tall(y):
    y4 = jnp.pad(y.reshape(_N, 4, 32).transpose(1, 0, 2),
                 ((0, 0), (0, _NP - _N), (0, 0)))
    return y4.reshape(4 * _NP, 32)


def _from_s4(s4):
    return s4[:, :_N, :].transpose(1, 0, 2).reshape(_N, 128)


def kernel(node_type, node_tactic_id, state_lm_id, edge_index, batch, type_emb,
           tactic_emb, state_lm_bank, W_proj, b_proj, W1_self, W1_neigh, W1_sem,
           b1, W2_self, W2_neigh, W2_sem, b2, Wc1, bc1, Wc2, bc2):
    t_type = jnp.take(type_emb, node_type, axis=0)
    shifted = jnp.clip(node_tactic_id + 1, 0, _NUM_TACTICS)
    t_tac = jnp.take(tactic_emb, shifted, axis=0)
    mask = state_lm_id >= 0
    safe_id = jnp.where(mask, state_lm_id, 0)
    proj_bank = state_lm_bank @ W_proj
    sem = jnp.where(mask[:, None], jnp.take(proj_bank, safe_id, axis=0) + b_proj, 0.0)
    x = jnp.concatenate([t_type, t_tac, sem], axis=-1)

    src, dst = edge_index[0], edge_index[1]
    srcp = jnp.pad(src, (0, _EP - _E))
    dstp_flat = jnp.pad(dst, (0, _EP - _E), constant_values=_N)
    src4 = (srcp[None, :] + (jnp.arange(4, dtype=jnp.int32) * _NP)[:, None]
            ).reshape(4, _ER, 128)
    dstp = dstp_flat.reshape(_ER, 128)
    zrows = jnp.zeros((_ZR, 32), jnp.float32)

    dhi = (dstp_flat >> 5).reshape(_ER, 128)
    dlo = (dstp_flat & 31).reshape(_ER, 128)
    deg_parts = _deg_count(dhi, dlo, jnp.eye(32, dtype=jnp.float32),
                           jnp.zeros((_DHT, 32), jnp.float32))
    deg = (deg_parts[0] + deg_parts[1]).reshape(_NP)[:_N]
    deg_inv = 1.0 / jnp.clip(deg, 1.0, None)

    y1 = x @ W1_neigh
    S1 = _from_s4(_edge_segsum(_to_tall(y1), src4, dstp, zrows))
    h1 = jax.nn.relu(x @ W1_self + S1 * deg_inv[:, None] + sem @ W1_sem + b1)
    y2 = h1 @ W2_neigh
    S2 = _from_s4(_edge_segsum(_to_tall(y2), src4, dstp, zrows))
    h2 = jax.nn.relu(h1 @ W2_self + S2 * deg_inv[:, None] + sem @ W2_sem + b2)

    sums = jax.ops.segment_sum(h2, batch, num_segments=_G)
    cnt = jax.ops.segment_sum(jnp.ones((_N,), jnp.float32), batch, num_segments=_G)
    graph_repr = sums / jnp.clip(cnt, 1.0, None)[:, None]

    return pl.pallas_call(
        _head_body,
        out_shape=jax.ShapeDtypeStruct((_G, _NUM_TACTICS), jnp.float32),
    )(graph_repr, Wc1, bc1, Wc2, bc2)


# speedup vs baseline: 2.2770x; 1.0146x over previous
import jax
import jax.numpy as jnp
from jax import lax
from jax.experimental import pallas as pl
from jax.experimental.pallas import tpu as pltpu
from jax.experimental.pallas import tpu_sc as plsc

_N = 50000
_E = 800000
_G = 256
_NUM_TACTICS = 1000

_NC, _NS = 2, 16
_NP = 50176
_EP = 819200
_ER = _EP // 128
_RPT = _ER // _NS
_BB = 4
_NB = _RPT // _BB
_ZR = _NP // _NS

_NPN = 53248
_NRW = _NPN // 128 // (_NC * _NS)


def _dot(a, b):
    return jnp.dot(a, b, precision=lax.Precision.HIGHEST)


def _edge_body(ytall, src4, dstp, zrows, s4, sidx, didx, rows, acc, gsem):
    c = lax.axis_index("c")
    s = lax.axis_index("s")
    for jj in range(2):
        chunk = c * 2 + jj
        pltpu.sync_copy(zrows, acc.at[pl.ds(s * _ZR, _ZR)])
        plsc.subcore_barrier()

        def batch_body(b, carry):
            row0 = s * _RPT + b * _BB
            pltpu.sync_copy(src4.at[chunk, pl.ds(row0, _BB), :], sidx)
            pltpu.sync_copy(dstp.at[pl.ds(row0, _BB), :], didx)
            cps = [
                pltpu.async_copy(ytall.at[sidx.at[j]],
                                 rows.at[pl.ds(j * 128, 128)], gsem)
                for j in range(_BB)
            ]
            for cp in cps:
                cp.wait()
            for j in range(_BB):
                pltpu.sync_copy(rows.at[pl.ds(j * 128, 128)],
                                acc.at[didx.at[j]], add=True)
            return carry

        lax.fori_loop(0, _NB, batch_body, 0)
        plsc.subcore_barrier()
        pltpu.sync_copy(acc.at[pl.ds(s * _ZR, _ZR)],
                        s4.at[chunk, pl.ds(s * _ZR, _ZR), :])
        plsc.subcore_barrier()


def _edge_segsum(ytall, src4, dstp, zrows):
    f = pl.kernel(
        _edge_body,
        out_type=jax.ShapeDtypeStruct((4, _NP, 32), jnp.float32),
        mesh=plsc.VectorSubcoreMesh(core_axis_name="c", subcore_axis_name="s",
                                    num_cores=_NC, num_subcores=_NS),
        scratch_types=[
            pltpu.VMEM((_BB, 128), jnp.int32),
            pltpu.VMEM((_BB, 128), jnp.int32),
            pltpu.VMEM((_BB * 128, 32), jnp.float32),
            pltpu.VMEM_SHARED((_NP, 32), jnp.float32),
            pltpu.SemaphoreType.DMA,
        ],
        compiler_params=pltpu.CompilerParams(use_tc_tiling_on_sc=False),
    )
    return f(ytall, src4, dstp, zrows)


_DB = 20
_DRW = _EP // 128 // (_NC * _NS)
_DNB = _DRW // _DB
_DH = _NP // 32
_DHT = _DH // _NS


def _deg_body(dhi, dlo, eye32, zrows_d, deg_out, hidx, lidx, rows, acc, gsem):
    c = lax.axis_index("c")
    s = lax.axis_index("s")
    w = s * _NC + c
    pltpu.sync_copy(zrows_d, acc.at[pl.ds(s * _DHT, _DHT)])
    plsc.subcore_barrier()

    def batch_body(b, carry):
        row0 = w * _DRW + b * _DB
        pltpu.sync_copy(dhi.at[pl.ds(row0, _DB), :], hidx)
        pltpu.sync_copy(dlo.at[pl.ds(row0, _DB), :], lidx)
        cps = [
            pltpu.async_copy(eye32.at[lidx.at[j]],
                             rows.at[pl.ds(j * 128, 128)], gsem)
            for j in range(_DB)
        ]
        for cp in cps:
            cp.wait()
        for j in range(_DB):
            pltpu.sync_copy(rows.at[pl.ds(j * 128, 128)],
                            acc.at[hidx.at[j]], add=True)
        return carry

    lax.fori_loop(0, _DNB, batch_body, 0)
    plsc.subcore_barrier()
    pltpu.sync_copy(acc.at[pl.ds(s * _DHT, _DHT)],
                    deg_out.at[c, pl.ds(s * _DHT, _DHT), :])


def _deg_count(dhi, dlo, eye32, zrows_d):
    f = pl.kernel(
        _deg_body,
        out_type=jax.ShapeDtypeStruct((_NC, _DH, 32), jnp.float32),
        mesh=plsc.VectorSubcoreMesh(core_axis_name="c", subcore_axis_name="s",
                                    num_cores=_NC, num_subcores=_NS),
        scratch_types=[
            pltpu.VMEM((_DB, 128), jnp.int32),
            pltpu.VMEM((_DB, 128), jnp.int32),
            pltpu.VMEM((_DB * 128, 32), jnp.float32),
            pltpu.VMEM_SHARED((_DH, 32), jnp.float32),
            pltpu.SemaphoreType.DMA,
        ],
        compiler_params=pltpu.CompilerParams(use_tc_tiling_on_sc=False),
    )
    return f(dhi, dlo, eye32, zrows_d)


def _node_body(t1s, tt1s, pbsz, t1n, tt1n, pbnz, pb2z, it_h, tt_h, pb_h,
               hs_out, y_out, s2_out, iti, tti, pbi, rhs, ry, rs2, gsem):
    c = lax.axis_index("c")
    s = lax.axis_index("s")
    w = s * _NC + c
    r0 = w * _NRW
    pltpu.sync_copy(it_h.at[pl.ds(r0, _NRW), :], iti)
    pltpu.sync_copy(tt_h.at[pl.ds(r0, _NRW), :], tti)
    pltpu.sync_copy(pb_h.at[pl.ds(r0, _NRW), :], pbi)

    def jb(j, carry):
        base = (r0 + j) * 128
        g1 = pltpu.async_copy(t1s.at[iti.at[j]], rhs, gsem)
        g2 = pltpu.async_copy(t1n.at[iti.at[j]], ry, gsem)
        g3 = pltpu.async_copy(pb2z.at[pbi.at[j]], rs2, gsem)
        g1.wait()
        g2.wait()
        g3.wait()
        a1 = pltpu.async_copy(tt1s.at[tti.at[j]], rhs, gsem, add=True)
        a2 = pltpu.async_copy(pbsz.at[pbi.at[j]], rhs, gsem, add=True)
        a3 = pltpu.async_copy(tt1n.at[tti.at[j]], ry, gsem, add=True)
        a4 = pltpu.async_copy(pbnz.at[pbi.at[j]], ry, gsem, add=True)
        a1.wait()
        a2.wait()
        a3.wait()
        a4.wait()
        pltpu.sync_copy(rhs, hs_out.at[pl.ds(base, 128)])
        pltpu.sync_copy(ry, y_out.at[pl.ds(base, 128)])
        pltpu.sync_copy(rs2, s2_out.at[pl.ds(base, 128)])
        return carry

    lax.fori_loop(0, _NRW, jb, 0)


def _node_gather(t1s, tt1s, pbsz, t1n, tt1n, pbnz, pb2z, it_h, tt_h, pb_h):
    f = pl.kernel(
        _node_body,
        out_type=(jax.ShapeDtypeStruct((_NPN, 128), jnp.float32),
                  jax.ShapeDtypeStruct((_NPN, 128), jnp.float32),
                  jax.ShapeDtypeStruct((_NPN, 128), jnp.float32)),
        mesh=plsc.VectorSubcoreMesh(core_axis_name="c", subcore_axis_name="s",
                                    num_cores=_NC, num_subcores=_NS),
        scratch_types=[
            pltpu.VMEM((_NRW, 128), jnp.int32),
            pltpu.VMEM((_NRW, 128), jnp.int32),
            pltpu.VMEM((_NRW, 128), jnp.int32),
            pltpu.VMEM((128, 128), jnp.float32),
            pltpu.VMEM((128, 128), jnp.float32),
            pltpu.VMEM((128, 128), jnp.float32),
            pltpu.SemaphoreType.DMA,
        ],
        compiler_params=pltpu.CompilerParams(use_tc_tiling_on_sc=False),
    )
    return f(t1s, tt1s, pbsz, t1n, tt1n, pbnz, pb2z, it_h, tt_h, pb_h)


def _fold_body(te, ta, wp, bp, w1s, w1n, w1sem, w2sem,
               t1s, t1n, tt1s, tt1n, wfs, wfn, wf2, cs, cn, c2):
    t1s[...] = _dot(te[...], w1s[0:32, :])
    t1n[...] = _dot(te[...], w1n[0:32, :])
    tt1s[...] = _dot(ta[...], w1s[32:96, :])
    tt1n[...] = _dot(ta[...], w1n[32:96, :])
    vs = w1s[96:160, :] + w1sem[...]
    vn = w1n[96:160, :]
    v2 = w2sem[...]
    wfs[...] = _dot(wp[...], vs)
    wfn[...] = _dot(wp[...], vn)
    wf2[...] = _dot(wp[...], v2)
    cs[...] = jnp.broadcast_to(_dot(bp[...], vs), (8, 128))
    cn[...] = jnp.broadcast_to(_dot(bp[...], vn), (8, 128))
    c2[...] = jnp.broadcast_to(_dot(bp[...], v2), (8, 128))


def _fold_tables(te, ta, wp, bp, w1s, w1n, w1sem, w2sem):
    outs = [jax.ShapeDtypeStruct(s, jnp.float32) for s in
            [(32, 128), (32, 128), (1008, 128), (1008, 128),
             (768, 128), (768, 128), (768, 128),
             (8, 128), (8, 128), (8, 128)]]
    return pl.pallas_call(_fold_body, out_shape=outs)(
        te, ta, wp, bp, w1s, w1n, w1sem, w2sem)


_BKP = 20096


def _bank_body(bank, wfs, wfn, wf2, cs, cn, c2, pbs, pbn, pb2):
    b = bank[...]
    pbs[...] = _dot(b, wfs[...]) + cs[0:1, :]
    pbn[...] = _dot(b, wfn[...]) + cn[0:1, :]
    pb2[...] = _dot(b, wf2[...]) + c2[0:1, :]


def _bank_tables(bank_pad, wfs, wfn, wf2, cs, cn, c2):
    nb = _BKP // 128
    blk = lambda i: (i, 0)
    full = lambda i: (0, 0)
    return pl.pallas_call(
        _bank_body,
        grid=(nb,),
        in_specs=[pl.BlockSpec((128, 768), blk),
                  pl.BlockSpec((768, 128), full), pl.BlockSpec((768, 128), full),
                  pl.BlockSpec((768, 128), full),
                  pl.BlockSpec((8, 128), full), pl.BlockSpec((8, 128), full),
                  pl.BlockSpec((8, 128), full)],
        out_specs=[pl.BlockSpec((128, 128), blk)] * 3,
        out_shape=[jax.ShapeDtypeStruct((_BKP, 128), jnp.float32)] * 3,
    )(bank_pad, wfs, wfn, wf2, cs, cn, c2)


def _mid_body(hs1, s1, sem2, degp, w2s, w2n, b1, b2, hs2, y2):
    deg = degp[0, :] + degp[1, :]
    di = 1.0 / jnp.maximum(deg, 1.0)
    h1 = jnp.maximum(hs1[...] + b1[...] + s1[...] * di[:, None], 0.0)
    hs2[...] = _dot(h1, w2s[...]) + sem2[...] + b2[...]
    y2[...] = _dot(h1, w2n[...])


def _mid(hs1, s1, sem2, degp, w2s, w2n, b1, b2):
    nb = _NP // 512
    blk = lambda i: (i, 0)
    full = lambda i: (0, 0)
    return pl.pallas_call(
        _mid_body,
        grid=(nb,),
        in_specs=[pl.BlockSpec((512, 128), blk), pl.BlockSpec((512, 128), blk),
                  pl.BlockSpec((512, 128), blk),
                  pl.BlockSpec((2, 512), lambda i: (0, i)),
                  pl.BlockSpec((128, 128), full), pl.BlockSpec((128, 128), full),
                  pl.BlockSpec((1, 128), full), pl.BlockSpec((1, 128), full)],
        out_specs=[pl.BlockSpec((512, 128), blk)] * 2,
        out_shape=[jax.ShapeDtypeStruct((_NP, 128), jnp.float32)] * 2,
    )(hs1, s1, sem2, degp, w2s, w2n, b1, b2)


def _pool_body(hs2, s2, degp, batchp, wc1, bc1, wc2, bc2, out, sums, cnt):
    pid = pl.program_id(0)
    nb = pl.num_programs(0)

    @pl.when(pid == 0)
    def _():
        sums[...] = jnp.zeros_like(sums)
        cnt[...] = jnp.zeros_like(cnt)

    deg = degp[0, :] + degp[1, :]
    di = 1.0 / jnp.maximum(deg, 1.0)
    h2 = jnp.maximum(hs2[...] + s2[...] * di[:, None], 0.0)
    bb = batchp[0, :]
    oh = (lax.broadcasted_iota(jnp.int32, (_G, 512), 0) == bb[None, :]
          ).astype(jnp.float32)
    sums[...] += _dot(oh, h2)
    cnt[...] += jnp.broadcast_to(jnp.sum(oh, axis=1, keepdims=True), (_G, 128))

    @pl.when(pid == nb - 1)
    def _():
        gr = sums[...] / jnp.maximum(cnt[...], 1.0)
        h = jnp.maximum(_dot(gr, wc1[...]) + bc1[...], 0.0)
        out[...] = _dot(h, wc2[...]) + bc2[...]


def _pool_head(hs2, s2, degp, batchp, wc1, bc1, wc2, bc2):
    nb = _NP // 512
    blk = lambda i: (i, 0)
    full = lambda i: (0, 0)
    return pl.pallas_call(
        _pool_body,
        grid=(nb,),
        in_specs=[pl.BlockSpec((512, 128), blk), pl.BlockSpec((512, 128), blk),
                  pl.BlockSpec((2, 512), lambda i: (0, i)),
                  pl.BlockSpec((1, 512), lambda i: (0, i)),
                  pl.BlockSpec((128, 128), full), pl.BlockSpec((1, 128), full),
                  pl.BlockSpec((128, _NUM_TACTICS), full),
                  pl.BlockSpec((1, _NUM_TACTICS), full)],
        out_specs=pl.BlockSpec((_G, _NUM_TACTICS), full),
        out_shape=jax.ShapeDtypeStruct((_G, _NUM_TACTICS), jnp.float32),
        scratch_shapes=[pltpu.VMEM((_G, 128), jnp.float32),
                        pltpu.VMEM((_G, 128), jnp.float32)],
    )(hs2, s2, degp, batchp, wc1, bc1, wc2, bc2)


def _to_tall(y):
    return y[:_NP].reshape(_NP, 4, 32).transpose(1, 0, 2).reshape(4 * _NP, 32)


def _from_s4(s4):
    return s4.transpose(1, 0, 2).reshape(_NP, 128)


def _pad_rows(a, n):
    return jnp.pad(a, ((0, n - a.shape[0]),) + ((0, 0),) * (a.ndim - 1))


def kernel(node_type, node_tactic_id, state_lm_id, edge_index, batch, type_emb,
           tactic_emb, state_lm_bank, W_proj, b_proj, W1_self, W1_neigh, W1_sem,
           b1, W2_self, W2_neigh, W2_sem, b2, Wc1, bc1, Wc2, bc2):
    ta_pad = _pad_rows(tactic_emb, 1008)
    (t1s, t1n, tt1s, tt1n, wfs, wfn, wf2, cs, cn, c2) = _fold_tables(
        type_emb, ta_pad, W_proj, b_proj.reshape(1, 64),
        W1_self, W1_neigh, W1_sem, W2_sem)
    bank_pad = _pad_rows(state_lm_bank, _BKP)
    pbs, pbn, pb2 = _bank_tables(bank_pad, wfs, wfn, wf2, cs, cn, c2)
    z8 = jnp.zeros((8, 128), jnp.float32)
    pbsz = jnp.concatenate([z8, pbs])
    pbnz = jnp.concatenate([z8, pbn])
    pb2z = jnp.concatenate([z8, pb2])

    mask = state_lm_id >= 0
    shifted = jnp.clip(node_tactic_id + 1, 0, _NUM_TACTICS)
    pb_idx = jnp.where(mask, 8 + jnp.where(mask, state_lm_id, 0), 0)
    it_h = jnp.pad(node_type, (0, _NPN - _N)).reshape(_NPN // 128, 128)
    tt_h = jnp.pad(shifted, (0, _NPN - _N)).reshape(_NPN // 128, 128)
    pb_h = jnp.pad(pb_idx, (0, _NPN - _N)).reshape(_NPN // 128, 128)

    hs1pre, y1, sem2c = _node_gather(t1s, tt1s, pbsz, t1n, tt1n, pbnz, pb2z,
                                     it_h, tt_h, pb_h)

    src, dst = edge_index[0], edge_index[1]
    srcp = jnp.pad(src, (0, _EP - _E))
    dstp_flat = jnp.pad(dst, (0, _EP - _E), constant_values=_N)
    src4 = (srcp[None, :] + (jnp.arange(4, dtype=jnp.int32) * _NP)[:, None]
            ).reshape(4, _ER, 128)
    dstp = dstp_flat.reshape(_ER, 128)
    zrows = jnp.zeros((_ZR, 32), jnp.float32)

    dhi = (dstp_flat >> 5).reshape(_ER, 128)
    dlo = (dstp_flat & 31).reshape(_ER, 128)
    deg_parts = _deg_count(dhi, dlo, jnp.eye(32, dtype=jnp.float32),
                           jnp.zeros((_DHT, 32), jnp.float32))
    degp = deg_parts.reshape(_NC, _NP)

    S1 = _from_s4(_edge_segsum(_to_tall(y1), src4, dstp, zrows))
    hs2pre, y2 = _mid(hs1pre[:_NP], S1, sem2c[:_NP], degp,
                      W2_self, W2_neigh, b1.reshape(1, 128), b2.reshape(1, 128))

    S2 = _from_s4(_edge_segsum(_to_tall(y2), src4, dstp, zrows))
    batchp = jnp.pad(batch, (0, _NP - _N), constant_values=_G
                     ).reshape(1, _NP)
    return _pool_head(hs2pre, S2, degp, batchp,
                      Wc1, bc1.reshape(1, 128), Wc2,
                      bc2.reshape(1, _NUM_TACTICS))


# speedup vs baseline: 2.4010x; 1.0545x over previous
import jax
import jax.numpy as jnp
from jax import lax
from jax.experimental import pallas as pl
from jax.experimental.pallas import tpu as pltpu
from jax.experimental.pallas import tpu_sc as plsc

_N = 50000
_E = 800000
_G = 256
_NUM_TACTICS = 1000

_NC, _NS = 2, 16
_NP = 50176
_EP = 819200
_ER = _EP // 128
_RPT = _ER // _NS
_BB = 4
_NB = _RPT // _BB
_ZR = _NP // _NS

_NPN = 53248
_NRW = _NPN // 128 // (_NC * _NS)


def _dot(a, b):
    return jnp.dot(a, b, precision=lax.Precision.HIGHEST)


def _edge_body(ytall, src4, dstp, zrows, s4, sidx, didx, rows, acc, gsem):
    c = lax.axis_index("c")
    s = lax.axis_index("s")
    for jj in range(2):
        chunk = c * 2 + jj
        pltpu.sync_copy(zrows, acc.at[pl.ds(s * _ZR, _ZR)])
        plsc.subcore_barrier()

        def batch_body(b, carry):
            row0 = s * _RPT + b * _BB
            pltpu.sync_copy(src4.at[chunk, pl.ds(row0, _BB), :], sidx)
            pltpu.sync_copy(dstp.at[pl.ds(row0, _BB), :], didx)
            cps = [
                pltpu.async_copy(ytall.at[sidx.at[j]],
                                 rows.at[pl.ds(j * 128, 128)], gsem)
                for j in range(_BB)
            ]
            for cp in cps:
                cp.wait()
            for j in range(_BB):
                pltpu.sync_copy(rows.at[pl.ds(j * 128, 128)],
                                acc.at[didx.at[j]], add=True)
            return carry

        lax.fori_loop(0, _NB, batch_body, 0)
        plsc.subcore_barrier()
        pltpu.sync_copy(acc.at[pl.ds(s * _ZR, _ZR)],
                        s4.at[chunk, pl.ds(s * _ZR, _ZR), :])
        plsc.subcore_barrier()


def _edge_segsum(ytall, src4, dstp, zrows):
    f = pl.kernel(
        _edge_body,
        out_type=jax.ShapeDtypeStruct((4, _NP, 32), jnp.float32),
        mesh=plsc.VectorSubcoreMesh(core_axis_name="c", subcore_axis_name="s",
                                    num_cores=_NC, num_subcores=_NS),
        scratch_types=[
            pltpu.VMEM((_BB, 128), jnp.int32),
            pltpu.VMEM((_BB, 128), jnp.int32),
            pltpu.VMEM((_BB * 128, 32), jnp.float32),
            pltpu.VMEM_SHARED((_NP, 32), jnp.float32),
            pltpu.SemaphoreType.DMA,
        ],
        compiler_params=pltpu.CompilerParams(use_tc_tiling_on_sc=False),
    )
    return f(ytall, src4, dstp, zrows)


_DB = 20
_DRW = _EP // 128 // (_NC * _NS)
_DNB = _DRW // _DB
_DH = _NP // 32
_DGT = 4 * _DH // _NS


def _deg_body(dhi4, dlo, eye32, zrows_d, deg_out, hidx, lidx, rows, acc, gsem):
    c = lax.axis_index("c")
    s = lax.axis_index("s")
    w = s * _NC + c
    g = s % 4
    pltpu.sync_copy(zrows_d, acc.at[pl.ds(s * _DGT, _DGT)])
    plsc.subcore_barrier()

    def batch_body(b, carry):
        row0 = w * _DRW + b * _DB
        pltpu.sync_copy(dhi4.at[g, pl.ds(row0, _DB), :], hidx)
        pltpu.sync_copy(dlo.at[pl.ds(row0, _DB), :], lidx)
        cps = [
            pltpu.async_copy(eye32.at[lidx.at[j]],
                             rows.at[pl.ds(j * 128, 128)], gsem)
            for j in range(_DB)
        ]
        for cp in cps:
            cp.wait()
        for j in range(_DB):
            pltpu.sync_copy(rows.at[pl.ds(j * 128, 128)],
                            acc.at[hidx.at[j]], add=True)
        return carry

    lax.fori_loop(0, _DNB, batch_body, 0)
    plsc.subcore_barrier()
    pltpu.sync_copy(acc.at[pl.ds(s * _DGT, _DGT)],
                    deg_out.at[c, pl.ds(s * _DGT, _DGT), :])


def _deg_count(dhi4, dlo, eye32, zrows_d):
    f = pl.kernel(
        _deg_body,
        out_type=jax.ShapeDtypeStruct((_NC, 4 * _DH, 32), jnp.float32),
        mesh=plsc.VectorSubcoreMesh(core_axis_name="c", subcore_axis_name="s",
                                    num_cores=_NC, num_subcores=_NS),
        scratch_types=[
            pltpu.VMEM((_DB, 128), jnp.int32),
            pltpu.VMEM((_DB, 128), jnp.int32),
            pltpu.VMEM((_DB * 128, 32), jnp.float32),
            pltpu.VMEM_SHARED((4 * _DH, 32), jnp.float32),
            pltpu.SemaphoreType.DMA,
        ],
        compiler_params=pltpu.CompilerParams(use_tc_tiling_on_sc=False),
    )
    return f(dhi4, dlo, eye32, zrows_d)


def _node_body(t1s, tt1s, pbsz, t1n, tt1n, pbnz, pb2z, it_h, tt_h, pb_h,
               hs_out, y_out, s2_out, iti, tti, pbi, rhs, ry, rs2, gsem):
    c = lax.axis_index("c")
    s = lax.axis_index("s")
    w = s * _NC + c
    r0 = w * _NRW
    pltpu.sync_copy(it_h.at[pl.ds(r0, _NRW), :], iti)
    pltpu.sync_copy(tt_h.at[pl.ds(r0, _NRW), :], tti)
    pltpu.sync_copy(pb_h.at[pl.ds(r0, _NRW), :], pbi)

    def jb(j, carry):
        base = (r0 + j) * 128
        g1 = pltpu.async_copy(t1s.at[iti.at[j]], rhs, gsem)
        g2 = pltpu.async_copy(t1n.at[iti.at[j]], ry, gsem)
        g3 = pltpu.async_copy(pb2z.at[pbi.at[j]], rs2, gsem)
        g1.wait()
        g2.wait()
        g3.wait()
        a1 = pltpu.async_copy(tt1s.at[tti.at[j]], rhs, gsem, add=True)
        a2 = pltpu.async_copy(pbsz.at[pbi.at[j]], rhs, gsem, add=True)
        a3 = pltpu.async_copy(tt1n.at[tti.at[j]], ry, gsem, add=True)
        a4 = pltpu.async_copy(pbnz.at[pbi.at[j]], ry, gsem, add=True)
        a1.wait()
        a2.wait()
        a3.wait()
        a4.wait()
        pltpu.sync_copy(rhs, hs_out.at[pl.ds(base, 128)])
        pltpu.sync_copy(ry, y_out.at[pl.ds(base, 128)])
        pltpu.sync_copy(rs2, s2_out.at[pl.ds(base, 128)])
        return carry

    lax.fori_loop(0, _NRW, jb, 0)


def _node_gather(t1s, tt1s, pbsz, t1n, tt1n, pbnz, pb2z, it_h, tt_h, pb_h):
    f = pl.kernel(
        _node_body,
        out_type=(jax.ShapeDtypeStruct((_NPN, 128), jnp.float32),
                  jax.ShapeDtypeStruct((_NPN, 128), jnp.float32),
                  jax.ShapeDtypeStruct((_NPN, 128), jnp.float32)),
        mesh=plsc.VectorSubcoreMesh(core_axis_name="c", subcore_axis_name="s",
                                    num_cores=_NC, num_subcores=_NS),
        scratch_types=[
            pltpu.VMEM((_NRW, 128), jnp.int32),
            pltpu.VMEM((_NRW, 128), jnp.int32),
            pltpu.VMEM((_NRW, 128), jnp.int32),
            pltpu.VMEM((128, 128), jnp.float32),
            pltpu.VMEM((128, 128), jnp.float32),
            pltpu.VMEM((128, 128), jnp.float32),
            pltpu.SemaphoreType.DMA,
        ],
        compiler_params=pltpu.CompilerParams(use_tc_tiling_on_sc=False),
    )
    return f(t1s, tt1s, pbsz, t1n, tt1n, pbnz, pb2z, it_h, tt_h, pb_h)


def _fold_body(te, ta, wp, bp, w1s, w1n, w1sem, w2sem,
               t1s, t1n, tt1s, tt1n, wfs, wfn, wf2, cs, cn, c2):
    t1s[...] = _dot(te[...], w1s[0:32, :])
    t1n[...] = _dot(te[...], w1n[0:32, :])
    tt1s[...] = _dot(ta[...], w1s[32:96, :])
    tt1n[...] = _dot(ta[...], w1n[32:96, :])
    vs = w1s[96:160, :] + w1sem[...]
    vn = w1n[96:160, :]
    v2 = w2sem[...]
    wfs[...] = _dot(wp[...], vs)
    wfn[...] = _dot(wp[...], vn)
    wf2[...] = _dot(wp[...], v2)
    cs[...] = jnp.broadcast_to(_dot(bp[...], vs), (8, 128))
    cn[...] = jnp.broadcast_to(_dot(bp[...], vn), (8, 128))
    c2[...] = jnp.broadcast_to(_dot(bp[...], v2), (8, 128))


def _fold_tables(te, ta, wp, bp, w1s, w1n, w1sem, w2sem):
    outs = [jax.ShapeDtypeStruct(s, jnp.float32) for s in
            [(32, 128), (32, 128), (1008, 128), (1008, 128),
             (768, 128), (768, 128), (768, 128),
             (8, 128), (8, 128), (8, 128)]]
    return pl.pallas_call(_fold_body, out_shape=outs)(
        te, ta, wp, bp, w1s, w1n, w1sem, w2sem)


_BKP = 20096


def _bank_body(bank, wfs, wfn, wf2, cs, cn, c2, pbs, pbn, pb2):
    b = bank[...]
    pbs[...] = _dot(b, wfs[...]) + cs[0:1, :]
    pbn[...] = _dot(b, wfn[...]) + cn[0:1, :]
    pb2[...] = _dot(b, wf2[...]) + c2[0:1, :]


def _bank_tables(bank_pad, wfs, wfn, wf2, cs, cn, c2):
    nb = _BKP // 128
    blk = lambda i: (i, 0)
    full = lambda i: (0, 0)
    return pl.pallas_call(
        _bank_body,
        grid=(nb,),
        in_specs=[pl.BlockSpec((128, 768), blk),
                  pl.BlockSpec((768, 128), full), pl.BlockSpec((768, 128), full),
                  pl.BlockSpec((768, 128), full),
                  pl.BlockSpec((8, 128), full), pl.BlockSpec((8, 128), full),
                  pl.BlockSpec((8, 128), full)],
        out_specs=[pl.BlockSpec((128, 128), blk)] * 3,
        out_shape=[jax.ShapeDtypeStruct((_BKP, 128), jnp.float32)] * 3,
    )(bank_pad, wfs, wfn, wf2, cs, cn, c2)


def _mid_body(hs1, s1, sem2, degp, w2s, w2n, b1, b2, hs2, y2):
    deg = jnp.sum(degp[...], axis=0)
    di = 1.0 / jnp.maximum(deg, 1.0)
    h1 = jnp.maximum(hs1[...] + b1[...] + s1[...] * di[:, None], 0.0)
    hs2[...] = _dot(h1, w2s[...]) + sem2[...] + b2[...]
    y2[...] = _dot(h1, w2n[...])


def _mid(hs1, s1, sem2, degp, w2s, w2n, b1, b2):
    nb = _NP // 512
    blk = lambda i: (i, 0)
    full = lambda i: (0, 0)
    return pl.pallas_call(
        _mid_body,
        grid=(nb,),
        in_specs=[pl.BlockSpec((512, 128), blk), pl.BlockSpec((512, 128), blk),
                  pl.BlockSpec((512, 128), blk),
                  pl.BlockSpec((8, 512), lambda i: (0, i)),
                  pl.BlockSpec((128, 128), full), pl.BlockSpec((128, 128), full),
                  pl.BlockSpec((1, 128), full), pl.BlockSpec((1, 128), full)],
        out_specs=[pl.BlockSpec((512, 128), blk)] * 2,
        out_shape=[jax.ShapeDtypeStruct((_NP, 128), jnp.float32)] * 2,
    )(hs1, s1, sem2, degp, w2s, w2n, b1, b2)


def _pool_body(hs2, s2, degp, batchp, wc1, bc1, wc2, bc2, out, sums, cnt):
    pid = pl.program_id(0)
    nb = pl.num_programs(0)

    @pl.when(pid == 0)
    def _():
        sums[...] = jnp.zeros_like(sums)
        cnt[...] = jnp.zeros_like(cnt)

    deg = jnp.sum(degp[...], axis=0)
    di = 1.0 / jnp.maximum(deg, 1.0)
    h2 = jnp.maximum(hs2[...] + s2[...] * di[:, None], 0.0)
    bb = batchp[0, :]
    oh = (lax.broadcasted_iota(jnp.int32, (_G, 512), 0) == bb[None, :]
          ).astype(jnp.float32)
    sums[...] += _dot(oh, h2)
    cnt[...] += jnp.broadcast_to(jnp.sum(oh, axis=1, keepdims=True), (_G, 128))

    @pl.when(pid == nb - 1)
    def _():
        gr = sums[...] / jnp.maximum(cnt[...], 1.0)
        h = jnp.maximum(_dot(gr, wc1[...]) + bc1[...], 0.0)
        out[...] = _dot(h, wc2[...]) + bc2[...]


def _pool_head(hs2, s2, degp, batchp, wc1, bc1, wc2, bc2):
    nb = _NP // 512
    blk = lambda i: (i, 0)
    full = lambda i: (0, 0)
    return pl.pallas_call(
        _pool_body,
        grid=(nb,),
        in_specs=[pl.BlockSpec((512, 128), blk), pl.BlockSpec((512, 128), blk),
                  pl.BlockSpec((8, 512), lambda i: (0, i)),
                  pl.BlockSpec((1, 512), lambda i: (0, i)),
                  pl.BlockSpec((128, 128), full), pl.BlockSpec((1, 128), full),
                  pl.BlockSpec((128, _NUM_TACTICS), full),
                  pl.BlockSpec((1, _NUM_TACTICS), full)],
        out_specs=pl.BlockSpec((_G, _NUM_TACTICS), full),
        out_shape=jax.ShapeDtypeStruct((_G, _NUM_TACTICS), jnp.float32),
        scratch_shapes=[pltpu.VMEM((_G, 128), jnp.float32),
                        pltpu.VMEM((_G, 128), jnp.float32)],
    )(hs2, s2, degp, batchp, wc1, bc1, wc2, bc2)


def _to_tall(y):
    return y[:_NP].reshape(_NP, 4, 32).transpose(1, 0, 2).reshape(4 * _NP, 32)


def _from_s4(s4):
    return s4.transpose(1, 0, 2).reshape(_NP, 128)


def _pad_rows(a, n):
    return jnp.pad(a, ((0, n - a.shape[0]),) + ((0, 0),) * (a.ndim - 1))


def kernel(node_type, node_tactic_id, state_lm_id, edge_index, batch, type_emb,
           tactic_emb, state_lm_bank, W_proj, b_proj, W1_self, W1_neigh, W1_sem,
           b1, W2_self, W2_neigh, W2_sem, b2, Wc1, bc1, Wc2, bc2):
    ta_pad = _pad_rows(tactic_emb, 1008)
    (t1s, t1n, tt1s, tt1n, wfs, wfn, wf2, cs, cn, c2) = _fold_tables(
        type_emb, ta_pad, W_proj, b_proj.reshape(1, 64),
        W1_self, W1_neigh, W1_sem, W2_sem)
    bank_pad = _pad_rows(state_lm_bank, _BKP)
    pbs, pbn, pb2 = _bank_tables(bank_pad, wfs, wfn, wf2, cs, cn, c2)
    z8 = jnp.zeros((8, 128), jnp.float32)
    pbsz = jnp.concatenate([z8, pbs])
    pbnz = jnp.concatenate([z8, pbn])
    pb2z = jnp.concatenate([z8, pb2])

    mask = state_lm_id >= 0
    shifted = jnp.clip(node_tactic_id + 1, 0, _NUM_TACTICS)
    pb_idx = jnp.where(mask, 8 + jnp.where(mask, state_lm_id, 0), 0)
    it_h = jnp.pad(node_type, (0, _NPN - _N)).reshape(_NPN // 128, 128)
    tt_h = jnp.pad(shifted, (0, _NPN - _N)).reshape(_NPN // 128, 128)
    pb_h = jnp.pad(pb_idx, (0, _NPN - _N)).reshape(_NPN // 128, 128)

    hs1pre, y1, sem2c = _node_gather(t1s, tt1s, pbsz, t1n, tt1n, pbnz, pb2z,
                                     it_h, tt_h, pb_h)

    src, dst = edge_index[0], edge_index[1]
    srcp = jnp.pad(src, (0, _EP - _E))
    dstp_flat = jnp.pad(dst, (0, _EP - _E), constant_values=_N)
    src4 = (srcp[None, :] + (jnp.arange(4, dtype=jnp.int32) * _NP)[:, None]
            ).reshape(4, _ER, 128)
    dstp = dstp_flat.reshape(_ER, 128)
    zrows = jnp.zeros((_ZR, 32), jnp.float32)

    dhi_flat = dstp_flat >> 5
    dhi4 = (dhi_flat[None, :] + (jnp.arange(4, dtype=jnp.int32) * _DH)[:, None]
            ).reshape(4, _ER, 128)
    dlo = (dstp_flat & 31).reshape(_ER, 128)
    deg_parts = _deg_count(dhi4, dlo, jnp.eye(32, dtype=jnp.float32),
                           jnp.zeros((_DGT, 32), jnp.float32))
    degp = deg_parts.reshape(_NC * 4, _NP)

    S1 = _from_s4(_edge_segsum(_to_tall(y1), src4, dstp, zrows))
    hs2pre, y2 = _mid(hs1pre[:_NP], S1, sem2c[:_NP], degp,
                      W2_self, W2_neigh, b1.reshape(1, 128), b2.reshape(1, 128))

    S2 = _from_s4(_edge_segsum(_to_tall(y2), src4, dstp, zrows))
    batchp = jnp.pad(batch, (0, _NP - _N), constant_values=_G
                     ).reshape(1, _NP)
    return _pool_head(hs2pre, S2, degp, batchp,
                      Wc1, bc1.reshape(1, 128), Wc2,
                      bc2.reshape(1, _NUM_TACTICS))


# speedup vs baseline: 2.4392x; 1.0159x over previous
import jax
import jax.numpy as jnp
from jax import lax
from jax.experimental import pallas as pl
from jax.experimental.pallas import tpu as pltpu
from jax.experimental.pallas import tpu_sc as plsc

_N = 50000
_E = 800000
_G = 256
_NUM_TACTICS = 1000

_NC, _NS = 2, 16
_NP = 50176
_EP = 819200
_ER = _EP // 128
_RPT = _ER // _NS
_BB = 4
_NB = _RPT // _BB
_ZR = _NP // _NS

_NPN = 53248
_NRW = _NPN // 128 // (_NC * _NS)


def _dot(a, b):
    return jnp.dot(a, b, precision=lax.Precision.HIGHEST)


def _edge_body(ytall, src4, dstp, zrows, s4, sidx, didx, rows, acc, gsem):
    c = lax.axis_index("c")
    s = lax.axis_index("s")
    for jj in range(2):
        chunk = c * 2 + jj
        pltpu.sync_copy(zrows, acc.at[pl.ds(s * _ZR, _ZR)])
        plsc.subcore_barrier()

        def batch_body(b, carry):
            row0 = s * _RPT + b * _BB
            pltpu.sync_copy(src4.at[chunk, pl.ds(row0, _BB), :], sidx)
            pltpu.sync_copy(dstp.at[pl.ds(row0, _BB), :], didx)
            cps = [
                pltpu.async_copy(ytall.at[sidx.at[j]],
                                 rows.at[pl.ds(j * 128, 128)], gsem)
                for j in range(_BB)
            ]
            for cp in cps:
                cp.wait()
            aps = [
                pltpu.async_copy(rows.at[pl.ds(j * 128, 128)],
                                 acc.at[didx.at[j]], gsem, add=True)
                for j in range(_BB)
            ]
            for ap in aps:
                ap.wait()
            return carry

        lax.fori_loop(0, _NB, batch_body, 0)
        plsc.subcore_barrier()
        pltpu.sync_copy(acc.at[pl.ds(s * _ZR, _ZR)],
                        s4.at[chunk, pl.ds(s * _ZR, _ZR), :])
        plsc.subcore_barrier()


def _edge_segsum(ytall, src4, dstp, zrows):
    f = pl.kernel(
        _edge_body,
        out_type=jax.ShapeDtypeStruct((4, _NP, 32), jnp.float32),
        mesh=plsc.VectorSubcoreMesh(core_axis_name="c", subcore_axis_name="s",
                                    num_cores=_NC, num_subcores=_NS),
        scratch_types=[
            pltpu.VMEM((_BB, 128), jnp.int32),
            pltpu.VMEM((_BB, 128), jnp.int32),
            pltpu.VMEM((_BB * 128, 32), jnp.float32),
            pltpu.VMEM_SHARED((_NP, 32), jnp.float32),
            pltpu.SemaphoreType.DMA,
        ],
        compiler_params=pltpu.CompilerParams(use_tc_tiling_on_sc=False),
    )
    return f(ytall, src4, dstp, zrows)


_DB = 20
_DRW = _EP // 128 // (_NC * _NS)
_DNB = _DRW // _DB
_DH = _NP // 32
_DGT = 4 * _DH // _NS


def _deg_body(dhi4, dlo, eye32, zrows_d, deg_out, hidx, lidx, rows, acc, gsem):
    c = lax.axis_index("c")
    s = lax.axis_index("s")
    w = s * _NC + c
    g = s % 4
    pltpu.sync_copy(zrows_d, acc.at[pl.ds(s * _DGT, _DGT)])
    plsc.subcore_barrier()

    def batch_body(b, carry):
        row0 = w * _DRW + b * _DB
        pltpu.sync_copy(dhi4.at[g, pl.ds(row0, _DB), :], hidx)
        pltpu.sync_copy(dlo.at[pl.ds(row0, _DB), :], lidx)
        cps = [
            pltpu.async_copy(eye32.at[lidx.at[j]],
                             rows.at[pl.ds(j * 128, 128)], gsem)
            for j in range(_DB)
        ]
        for cp in cps:
            cp.wait()
        aps = [
            pltpu.async_copy(rows.at[pl.ds(j * 128, 128)],
                             acc.at[hidx.at[j]], gsem, add=True)
            for j in range(_DB)
        ]
        for ap in aps:
            ap.wait()
        return carry

    lax.fori_loop(0, _DNB, batch_body, 0)
    plsc.subcore_barrier()
    pltpu.sync_copy(acc.at[pl.ds(s * _DGT, _DGT)],
                    deg_out.at[c, pl.ds(s * _DGT, _DGT), :])


def _deg_count(dhi4, dlo, eye32, zrows_d):
    f = pl.kernel(
        _deg_body,
        out_type=jax.ShapeDtypeStruct((_NC, 4 * _DH, 32), jnp.float32),
        mesh=plsc.VectorSubcoreMesh(core_axis_name="c", subcore_axis_name="s",
                                    num_cores=_NC, num_subcores=_NS),
        scratch_types=[
            pltpu.VMEM((_DB, 128), jnp.int32),
            pltpu.VMEM((_DB, 128), jnp.int32),
            pltpu.VMEM((_DB * 128, 32), jnp.float32),
            pltpu.VMEM_SHARED((4 * _DH, 32), jnp.float32),
            pltpu.SemaphoreType.DMA,
        ],
        compiler_params=pltpu.CompilerParams(use_tc_tiling_on_sc=False),
    )
    return f(dhi4, dlo, eye32, zrows_d)


def _node_body(t1s, tt1s, pbsz, t1n, tt1n, pbnz, pb2z, it_h, tt_h, pb_h,
               hs_out, y_out, s2_out, iti, tti, pbi, rhs, ry, rs2, gsem):
    c = lax.axis_index("c")
    s = lax.axis_index("s")
    w = s * _NC + c
    r0 = w * _NRW
    pltpu.sync_copy(it_h.at[pl.ds(r0, _NRW), :], iti)
    pltpu.sync_copy(tt_h.at[pl.ds(r0, _NRW), :], tti)
    pltpu.sync_copy(pb_h.at[pl.ds(r0, _NRW), :], pbi)

    def jb(j, carry):
        base = (r0 + j) * 128
        g1 = pltpu.async_copy(t1s.at[iti.at[j]], rhs, gsem)
        g2 = pltpu.async_copy(t1n.at[iti.at[j]], ry, gsem)
        g3 = pltpu.async_copy(pb2z.at[pbi.at[j]], rs2, gsem)
        g1.wait()
        g2.wait()
        g3.wait()
        a1 = pltpu.async_copy(tt1s.at[tti.at[j]], rhs, gsem, add=True)
        a2 = pltpu.async_copy(pbsz.at[pbi.at[j]], rhs, gsem, add=True)
        a3 = pltpu.async_copy(tt1n.at[tti.at[j]], ry, gsem, add=True)
        a4 = pltpu.async_copy(pbnz.at[pbi.at[j]], ry, gsem, add=True)
        a1.wait()
        a2.wait()
        a3.wait()
        a4.wait()
        pltpu.sync_copy(rhs, hs_out.at[pl.ds(base, 128)])
        pltpu.sync_copy(ry, y_out.at[pl.ds(base, 128)])
        pltpu.sync_copy(rs2, s2_out.at[pl.ds(base, 128)])
        return carry

    lax.fori_loop(0, _NRW, jb, 0)


def _node_gather(t1s, tt1s, pbsz, t1n, tt1n, pbnz, pb2z, it_h, tt_h, pb_h):
    f = pl.kernel(
        _node_body,
        out_type=(jax.ShapeDtypeStruct((_NPN, 128), jnp.float32),
                  jax.ShapeDtypeStruct((_NPN, 128), jnp.float32),
                  jax.ShapeDtypeStruct((_NPN, 128), jnp.float32)),
        mesh=plsc.VectorSubcoreMesh(core_axis_name="c", subcore_axis_name="s",
                                    num_cores=_NC, num_subcores=_NS),
        scratch_types=[
            pltpu.VMEM((_NRW, 128), jnp.int32),
            pltpu.VMEM((_NRW, 128), jnp.int32),
            pltpu.VMEM((_NRW, 128), jnp.int32),
            pltpu.VMEM((128, 128), jnp.float32),
            pltpu.VMEM((128, 128), jnp.float32),
            pltpu.VMEM((128, 128), jnp.float32),
            pltpu.SemaphoreType.DMA,
        ],
        compiler_params=pltpu.CompilerParams(use_tc_tiling_on_sc=False),
    )
    return f(t1s, tt1s, pbsz, t1n, tt1n, pbnz, pb2z, it_h, tt_h, pb_h)


def _fold_body(te, ta, wp, bp, w1s, w1n, w1sem, w2sem,
               t1s, t1n, tt1s, tt1n, wfs, wfn, wf2, cs, cn, c2):
    t1s[...] = _dot(te[...], w1s[0:32, :])
    t1n[...] = _dot(te[...], w1n[0:32, :])
    tt1s[...] = _dot(ta[...], w1s[32:96, :])
    tt1n[...] = _dot(ta[...], w1n[32:96, :])
    vs = w1s[96:160, :] + w1sem[...]
    vn = w1n[96:160, :]
    v2 = w2sem[...]
    wfs[...] = _dot(wp[...], vs)
    wfn[...] = _dot(wp[...], vn)
    wf2[...] = _dot(wp[...], v2)
    cs[...] = jnp.broadcast_to(_dot(bp[...], vs), (8, 128))
    cn[...] = jnp.broadcast_to(_dot(bp[...], vn), (8, 128))
    c2[...] = jnp.broadcast_to(_dot(bp[...], v2), (8, 128))


def _fold_tables(te, ta, wp, bp, w1s, w1n, w1sem, w2sem):
    outs = [jax.ShapeDtypeStruct(s, jnp.float32) for s in
            [(32, 128), (32, 128), (1008, 128), (1008, 128),
             (768, 128), (768, 128), (768, 128),
             (8, 128), (8, 128), (8, 128)]]
    return pl.pallas_call(_fold_body, out_shape=outs)(
        te, ta, wp, bp, w1s, w1n, w1sem, w2sem)


_BKP = 20096


def _bank_body(bank, wfs, wfn, wf2, cs, cn, c2, pbs, pbn, pb2):
    b = bank[...]
    pbs[...] = _dot(b, wfs[...]) + cs[0:1, :]
    pbn[...] = _dot(b, wfn[...]) + cn[0:1, :]
    pb2[...] = _dot(b, wf2[...]) + c2[0:1, :]


def _bank_tables(bank_pad, wfs, wfn, wf2, cs, cn, c2):
    nb = _BKP // 128
    blk = lambda i: (i, 0)
    full = lambda i: (0, 0)
    return pl.pallas_call(
        _bank_body,
        grid=(nb,),
        in_specs=[pl.BlockSpec((128, 768), blk),
                  pl.BlockSpec((768, 128), full), pl.BlockSpec((768, 128), full),
                  pl.BlockSpec((768, 128), full),
                  pl.BlockSpec((8, 128), full), pl.BlockSpec((8, 128), full),
                  pl.BlockSpec((8, 128), full)],
        out_specs=[pl.BlockSpec((128, 128), blk)] * 3,
        out_shape=[jax.ShapeDtypeStruct((_BKP, 128), jnp.float32)] * 3,
    )(bank_pad, wfs, wfn, wf2, cs, cn, c2)


def _mid_body(hs1, s1, sem2, degp, w2s, w2n, b1, b2, hs2, y2):
    deg = jnp.sum(degp[...], axis=0)
    di = 1.0 / jnp.maximum(deg, 1.0)
    h1 = jnp.maximum(hs1[...] + b1[...] + s1[...] * di[:, None], 0.0)
    hs2[...] = _dot(h1, w2s[...]) + sem2[...] + b2[...]
    y2[...] = _dot(h1, w2n[...])


def _mid(hs1, s1, sem2, degp, w2s, w2n, b1, b2):
    nb = _NP // 512
    blk = lambda i: (i, 0)
    full = lambda i: (0, 0)
    return pl.pallas_call(
        _mid_body,
        grid=(nb,),
        in_specs=[pl.BlockSpec((512, 128), blk), pl.BlockSpec((512, 128), blk),
                  pl.BlockSpec((512, 128), blk),
                  pl.BlockSpec((8, 512), lambda i: (0, i)),
                  pl.BlockSpec((128, 128), full), pl.BlockSpec((128, 128), full),
                  pl.BlockSpec((1, 128), full), pl.BlockSpec((1, 128), full)],
        out_specs=[pl.BlockSpec((512, 128), blk)] * 2,
        out_shape=[jax.ShapeDtypeStruct((_NP, 128), jnp.float32)] * 2,
    )(hs1, s1, sem2, degp, w2s, w2n, b1, b2)


def _pool_body(hs2, s2, degp, batchp, wc1, bc1, wc2, bc2, out, sums, cnt):
    pid = pl.program_id(0)
    nb = pl.num_programs(0)

    @pl.when(pid == 0)
    def _():
        sums[...] = jnp.zeros_like(sums)
        cnt[...] = jnp.zeros_like(cnt)

    deg = jnp.sum(degp[...], axis=0)
    di = 1.0 / jnp.maximum(deg, 1.0)
    h2 = jnp.maximum(hs2[...] + s2[...] * di[:, None], 0.0)
    bb = batchp[0, :]
    oh = (lax.broadcasted_iota(jnp.int32, (_G, 512), 0) == bb[None, :]
          ).astype(jnp.float32)
    sums[...] += _dot(oh, h2)
    cnt[...] += jnp.broadcast_to(jnp.sum(oh, axis=1, keepdims=True), (_G, 128))

    @pl.when(pid == nb - 1)
    def _():
        gr = sums[...] / jnp.maximum(cnt[...], 1.0)
        h = jnp.maximum(_dot(gr, wc1[...]) + bc1[...], 0.0)
        out[...] = _dot(h, wc2[...]) + bc2[...]


def _pool_head(hs2, s2, degp, batchp, wc1, bc1, wc2, bc2):
    nb = _NP // 512
    blk = lambda i: (i, 0)
    full = lambda i: (0, 0)
    return pl.pallas_call(
        _pool_body,
        grid=(nb,),
        in_specs=[pl.BlockSpec((512, 128), blk), pl.BlockSpec((512, 128), blk),
                  pl.BlockSpec((8, 512), lambda i: (0, i)),
                  pl.BlockSpec((1, 512), lambda i: (0, i)),
                  pl.BlockSpec((128, 128), full), pl.BlockSpec((1, 128), full),
                  pl.BlockSpec((128, _NUM_TACTICS), full),
                  pl.BlockSpec((1, _NUM_TACTICS), full)],
        out_specs=pl.BlockSpec((_G, _NUM_TACTICS), full),
        out_shape=jax.ShapeDtypeStruct((_G, _NUM_TACTICS), jnp.float32),
        scratch_shapes=[pltpu.VMEM((_G, 128), jnp.float32),
                        pltpu.VMEM((_G, 128), jnp.float32)],
    )(hs2, s2, degp, batchp, wc1, bc1, wc2, bc2)


def _to_tall(y):
    return y[:_NP].reshape(_NP, 4, 32).transpose(1, 0, 2).reshape(4 * _NP, 32)


def _from_s4(s4):
    return s4.transpose(1, 0, 2).reshape(_NP, 128)


def _pad_rows(a, n):
    return jnp.pad(a, ((0, n - a.shape[0]),) + ((0, 0),) * (a.ndim - 1))


def kernel(node_type, node_tactic_id, state_lm_id, edge_index, batch, type_emb,
           tactic_emb, state_lm_bank, W_proj, b_proj, W1_self, W1_neigh, W1_sem,
           b1, W2_self, W2_neigh, W2_sem, b2, Wc1, bc1, Wc2, bc2):
    ta_pad = _pad_rows(tactic_emb, 1008)
    (t1s, t1n, tt1s, tt1n, wfs, wfn, wf2, cs, cn, c2) = _fold_tables(
        type_emb, ta_pad, W_proj, b_proj.reshape(1, 64),
        W1_self, W1_neigh, W1_sem, W2_sem)
    bank_pad = _pad_rows(state_lm_bank, _BKP)
    pbs, pbn, pb2 = _bank_tables(bank_pad, wfs, wfn, wf2, cs, cn, c2)
    z8 = jnp.zeros((8, 128), jnp.float32)
    pbsz = jnp.concatenate([z8, pbs])
    pbnz = jnp.concatenate([z8, pbn])
    pb2z = jnp.concatenate([z8, pb2])

    mask = state_lm_id >= 0
    shifted = jnp.clip(node_tactic_id + 1, 0, _NUM_TACTICS)
    pb_idx = jnp.where(mask, 8 + jnp.where(mask, state_lm_id, 0), 0)
    it_h = jnp.pad(node_type, (0, _NPN - _N)).reshape(_NPN // 128, 128)
    tt_h = jnp.pad(shifted, (0, _NPN - _N)).reshape(_NPN // 128, 128)
    pb_h = jnp.pad(pb_idx, (0, _NPN - _N)).reshape(_NPN // 128, 128)

    hs1pre, y1, sem2c = _node_gather(t1s, tt1s, pbsz, t1n, tt1n, pbnz, pb2z,
                                     it_h, tt_h, pb_h)

    src, dst = edge_index[0], edge_index[1]
    srcp = jnp.pad(src, (0, _EP - _E))
    dstp_flat = jnp.pad(dst, (0, _EP - _E), constant_values=_N)
    src4 = (srcp[None, :] + (jnp.arange(4, dtype=jnp.int32) * _NP)[:, None]
            ).reshape(4, _ER, 128)
    dstp = dstp_flat.reshape(_ER, 128)
    zrows = jnp.zeros((_ZR, 32), jnp.float32)

    dhi_flat = dstp_flat >> 5
    dhi4 = (dhi_flat[None, :] + (jnp.arange(4, dtype=jnp.int32) * _DH)[:, None]
            ).reshape(4, _ER, 128)
    dlo = (dstp_flat & 31).reshape(_ER, 128)
    deg_parts = _deg_count(dhi4, dlo, jnp.eye(32, dtype=jnp.float32),
                           jnp.zeros((_DGT, 32), jnp.float32))
    degp = deg_parts.reshape(_NC * 4, _NP)

    S1 = _from_s4(_edge_segsum(_to_tall(y1), src4, dstp, zrows))
    hs2pre, y2 = _mid(hs1pre[:_NP], S1, sem2c[:_NP], degp,
                      W2_self, W2_neigh, b1.reshape(1, 128), b2.reshape(1, 128))

    S2 = _from_s4(_edge_segsum(_to_tall(y2), src4, dstp, zrows))
    batchp = jnp.pad(batch, (0, _NP - _N), constant_values=_G
                     ).reshape(1, _NP)
    return _pool_head(hs2pre, S2, degp, batchp,
                      Wc1, bc1.reshape(1, 128), Wc2,
                      bc2.reshape(1, _NUM_TACTICS))
